# Initial kernel scaffold; baseline (speedup 1.0000x reference)
import jax
import jax.numpy as jnp
from jax.experimental import pallas as pl


def kernel(daytime, W_day, W_time, W_node):
    raise NotImplementedError("write your pallas kernel here")


# speedup vs baseline: 2.1704x; 2.1704x over previous
import jax
import jax.numpy as jnp
from jax.experimental import pallas as pl
from jax.experimental.pallas import tpu as pltpu

DAY_SIZE = 32
TIME_SIZE = 32


def _embed_block_kernel(idx_ref, wday_ref, wtime_ref, wnode_ref, out_ref):
    i = pl.program_id(0)
    d = idx_ref[i, 0]
    t = idx_ref[i, 1]
    node_count = out_ref.shape[1]
    day_row = wday_ref[pl.ds(d, 1), :]
    time_row = wtime_ref[pl.ds(t, 1), :]
    block = jnp.concatenate(
        (
            jnp.broadcast_to(day_row, (node_count, DAY_SIZE)),
            jnp.broadcast_to(time_row, (node_count, TIME_SIZE)),
            wnode_ref[...],
        ),
        axis=-1,
    )
    out_ref[0] = block


def kernel(daytime, W_day, W_time, W_node):
    batch, seq, _ = daytime.shape
    node_count, node_size = W_node.shape
    bs = batch * seq
    feat = DAY_SIZE + TIME_SIZE + node_size
    idx = daytime.reshape(bs, 2)

    grid_spec = pltpu.PrefetchScalarGridSpec(
        num_scalar_prefetch=1,
        grid=(bs,),
        in_specs=[
            pl.BlockSpec(W_day.shape, lambda i, idx_ref: (0, 0)),
            pl.BlockSpec(W_time.shape, lambda i, idx_ref: (0, 0)),
            pl.BlockSpec(W_node.shape, lambda i, idx_ref: (0, 0)),
        ],
        out_specs=pl.BlockSpec((1, node_count, feat), lambda i, idx_ref: (i, 0, 0)),
    )
    out = pl.pallas_call(
        _embed_block_kernel,
        grid_spec=grid_spec,
        out_shape=jax.ShapeDtypeStruct((bs, node_count, feat), jnp.float32),
    )(idx, W_day, W_time, W_node)
    return out.reshape(batch, seq, node_count, feat)


# speedup vs baseline: 6.2768x; 2.8920x over previous
import jax
import jax.numpy as jnp
from jax.experimental import pallas as pl
from jax.experimental.pallas import tpu as pltpu

DAY_SIZE = 32
TIME_SIZE = 32


BS = 8


def _embed_block_kernel(idx_ref, wday_ref, wtime_ref, wnode_ref, out_ref):
    g = pl.program_id(0)
    node_count = out_ref.shape[1]
    node_part = wnode_ref[...]
    for j in range(BS):
        i = g * BS + j
        d = idx_ref[i, 0]
        t = idx_ref[i, 1]
        day_row = wday_ref[pl.ds(d, 1), :]
        time_row = wtime_ref[pl.ds(t, 1), :]
        block = jnp.concatenate(
            (
                jnp.broadcast_to(day_row, (node_count, DAY_SIZE)),
                jnp.broadcast_to(time_row, (node_count, TIME_SIZE)),
                node_part,
            ),
            axis=-1,
        )
        out_ref[j] = block


def kernel(daytime, W_day, W_time, W_node):
    batch, seq, _ = daytime.shape
    node_count, node_size = W_node.shape
    bs = batch * seq
    feat = DAY_SIZE + TIME_SIZE + node_size
    idx = daytime.reshape(bs, 2)

    grid_spec = pltpu.PrefetchScalarGridSpec(
        num_scalar_prefetch=1,
        grid=(bs // BS,),
        in_specs=[
            pl.BlockSpec(W_day.shape, lambda i, idx_ref: (0, 0)),
            pl.BlockSpec(W_time.shape, lambda i, idx_ref: (0, 0)),
            pl.BlockSpec(W_node.shape, lambda i, idx_ref: (0, 0)),
        ],
        out_specs=pl.BlockSpec((BS, node_count, feat), lambda i, idx_ref: (i, 0, 0)),
    )
    out = pl.pallas_call(
        _embed_block_kernel,
        grid_spec=grid_spec,
        out_shape=jax.ShapeDtypeStruct((bs, node_count, feat), jnp.float32),
    )(idx, W_day, W_time, W_node)
    return out.reshape(batch, seq, node_count, feat)
